# baseline (speedup 1.0000x reference)
import functools

import jax
import jax.numpy as jnp
from jax import lax
from jax.experimental import pallas as pl
from jax.experimental.pallas import tpu as pltpu
from jax.experimental.pallas import tpu_sc as plsc

B, L, V, D = 16384, 200, 1000000, 64
NC, NS = 2, 16
NW = NC * NS
BPW = B // NW
C = 2
R = C * L
NCH = BPW // C
OUTROWS = 64
FLUSH = OUTROWS // C
STRIP = R
STRIPS = []
_off = 0
while _off < R:
    n = min(STRIP, R - _off)
    STRIPS.append((_off, n))
    _off += n

_KCOLS = D // 16


def _sc_bow_kernel(emb_hbm, xflat_hbm, out_hbm,
                   idx0, idx1, rows0, rows1, outv,
                   is0, is1, rs0, rs1):
    cid = lax.axis_index("c")
    sid = lax.axis_index("s")
    wid = sid * NC + cid
    base = wid * BPW

    idxb = [idx0, idx1]
    rowsb = [rows0, rows1]
    isem = [is0, is1]
    rsem = [rs0, rs1]

    def idx_start(g, b):
        start = pl.multiple_of((base + g * C) * L, R)
        pltpu.async_copy(xflat_hbm.at[pl.ds(start, R)], idxb[b], isem[b])

    def idx_wait(b):
        pltpu.make_async_copy(xflat_hbm.at[pl.ds(0, R)], idxb[b],
                              isem[b]).wait()

    def gathers_start(b):
        for (off, n) in STRIPS:
            pltpu.async_copy(emb_hbm.at[idxb[b].at[pl.ds(off, n)]],
                             rowsb[b].at[pl.ds(off, n)], rsem[b])

    def gathers_wait(b):
        for (off, n) in STRIPS:
            pltpu.make_async_copy(emb_hbm.at[idxb[b].at[pl.ds(off, n)]],
                                  rowsb[b].at[pl.ds(off, n)],
                                  rsem[b]).wait()

    def accumulate(g, b):
        rv = rowsb[b]

        def body(l, acc):
            new = []
            for r in range(C):
                for k in range(_KCOLS):
                    v = rv[r * L + l, pl.ds(k * 16, 16)]
                    new.append(acc[r * _KCOLS + k] + v)
            return tuple(new)

        zeros = tuple(jnp.zeros((16,), jnp.float32)
                      for _ in range(C * _KCOLS))
        acc = lax.fori_loop(0, L, body, zeros)
        orow0 = (g % FLUSH) * C
        for r in range(C):
            for k in range(_KCOLS):
                outv[orow0 + r, pl.ds(k * 16, 16)] = acc[r * _KCOLS + k]

    idx_start(0, 0)
    idx_wait(0)
    gathers_start(0)
    idx_start(1, 1)

    def outer(i, _):
        for b in range(2):
            g = i * 2 + b
            gathers_wait(b)

            @pl.when(g + 2 < NCH)
            def _():
                idx_start(g + 2, b)

            @pl.when(g + 1 < NCH)
            def _():
                idx_wait(1 - b)
                gathers_start(1 - b)

            accumulate(g, b)

            @pl.when(g % FLUSH == FLUSH - 1)
            def _():
                first = pl.multiple_of(base + (g + 1 - FLUSH) * C, OUTROWS)
                pltpu.sync_copy(outv, out_hbm.at[pl.ds(first, OUTROWS)])
        return 0

    lax.fori_loop(0, NCH // 2, outer, 0)


@jax.jit
def _sc_bow(emb, xflat):
    mesh = plsc.VectorSubcoreMesh(core_axis_name="c", subcore_axis_name="s",
                                  num_cores=NC, num_subcores=NS)
    f = pl.kernel(
        _sc_bow_kernel,
        out_type=jax.ShapeDtypeStruct((B, D), jnp.float32),
        mesh=mesh,
        scratch_types=[
            pltpu.VMEM((R,), jnp.int32),
            pltpu.VMEM((R,), jnp.int32),
            pltpu.VMEM((R, D), jnp.float32),
            pltpu.VMEM((R, D), jnp.float32),
            pltpu.VMEM((OUTROWS, D), jnp.float32),
            pltpu.SemaphoreType.DMA,
            pltpu.SemaphoreType.DMA,
            pltpu.SemaphoreType.DMA,
            pltpu.SemaphoreType.DMA,
        ],
        compiler_params=pltpu.CompilerParams(use_tc_tiling_on_sc=False),
    )
    return f(emb, xflat)


def _tc_head_kernel(sums_ref, t_ref, W1_ref, b1_ref, gamma_ref, beta_ref,
                    W2_ref, b2_ref, logits_ref, loss_ref):
    bow = sums_ref[...] * (1.0 / L)
    h = lax.dot_general(bow, W1_ref[...], (((1,), (1,)), ((), ())),
                        preferred_element_type=jnp.float32) + b1_ref[...]
    mu = jnp.mean(h, axis=0, keepdims=True)
    d = h - mu
    var = jnp.mean(d * d, axis=0, keepdims=True)
    hn = d * lax.rsqrt(var + 1e-5) * gamma_ref[...] + beta_ref[...]
    hr = jnp.maximum(hn, 0.0)
    logits = lax.dot_general(hr, W2_ref[...], (((1,), (0,)), ((), ())),
                             preferred_element_type=jnp.float32) + b2_ref[...]
    logits_ref[...] = logits
    t = t_ref[...]
    lv = (jnp.maximum(logits, 0.0) - logits * t
          + jnp.log1p(jnp.exp(-jnp.abs(logits))))
    loss_ref[...] = jnp.broadcast_to(jnp.mean(lv), (1, 1))


@jax.jit
def _tc_head(sums, t, W1, b1, gamma, beta, W2, b2):
    return pl.pallas_call(
        _tc_head_kernel,
        out_shape=(
            jax.ShapeDtypeStruct((B, 1), jnp.float32),
            jax.ShapeDtypeStruct((1, 1), jnp.float32),
        ),
    )(sums, t.reshape(B, 1), W1, b1.reshape(1, D), gamma.reshape(1, D),
      beta.reshape(1, D), W2, b2.reshape(1, 1))


def kernel(x, t, emb, W1, b1, gamma, beta, W2, b2):
    xflat = x.astype(jnp.int32).reshape(B * L)
    sums = _sc_bow(emb, xflat)
    logits2, loss2 = _tc_head(sums, t, W1, b1, gamma, beta, W2, b2)
    return (loss2[0, 0], logits2[:, 0])


# speedup vs baseline: 1.0028x; 1.0028x over previous
import functools

import jax
import jax.numpy as jnp
from jax import lax
from jax.experimental import pallas as pl
from jax.experimental.pallas import tpu as pltpu
from jax.experimental.pallas import tpu_sc as plsc

B, L, V, D = 16384, 200, 1000000, 64
NC, NS = 2, 16
NW = NC * NS
BPW = B // NW
C = 2
R = C * L
NCH = BPW // C
OUTROWS = 64
FLUSH = OUTROWS // C
STRIP = R
STRIPS = []
_off = 0
while _off < R:
    n = min(STRIP, R - _off)
    STRIPS.append((_off, n))
    _off += n

_KCOLS = D // 16


def _sc_bow_kernel(emb_hbm, xflat_hbm, out_hbm,
                   idx0, idx1, rows0, rows1, outv,
                   is0, is1, rs0, rs1):
    cid = lax.axis_index("c")
    sid = lax.axis_index("s")
    wid = sid * NC + cid
    base = wid * BPW

    idxb = [idx0, idx1]
    rowsb = [rows0, rows1]
    isem = [is0, is1]
    rsem = [rs0, rs1]

    def idx_start(g, b):
        start = pl.multiple_of((base + g * C) * L, R)
        pltpu.async_copy(xflat_hbm.at[pl.ds(start, R)], idxb[b], isem[b])

    def idx_wait(b):
        pltpu.make_async_copy(xflat_hbm.at[pl.ds(0, R)], idxb[b],
                              isem[b]).wait()

    def gathers_start(b):
        for (off, n) in STRIPS:
            pltpu.async_copy(emb_hbm.at[idxb[b].at[pl.ds(off, n)]],
                             rowsb[b].at[pl.ds(off, n)], rsem[b])

    def gathers_wait(b):
        for (off, n) in STRIPS:
            pltpu.make_async_copy(emb_hbm.at[idxb[b].at[pl.ds(off, n)]],
                                  rowsb[b].at[pl.ds(off, n)],
                                  rsem[b]).wait()

    def accumulate(g, b):
        rv = rowsb[b]
        zeros = tuple(jnp.zeros((16,), jnp.float32)
                      for _ in range(C * _KCOLS))

        @plsc.parallel_loop(0, L, step=1, unroll=4, carry=zeros)
        def acc(l, a):
            new = []
            for r in range(C):
                for k in range(_KCOLS):
                    v = rv[r * L + l, pl.ds(k * 16, 16)]
                    new.append(a[r * _KCOLS + k] + v)
            return tuple(new)
        orow0 = (g % FLUSH) * C
        for r in range(C):
            for k in range(_KCOLS):
                outv[orow0 + r, pl.ds(k * 16, 16)] = acc[r * _KCOLS + k]

    idx_start(0, 0)
    idx_wait(0)
    gathers_start(0)
    idx_start(1, 1)

    def outer(i, _):
        for b in range(2):
            g = i * 2 + b
            gathers_wait(b)

            @pl.when(g + 2 < NCH)
            def _():
                idx_start(g + 2, b)

            @pl.when(g + 1 < NCH)
            def _():
                idx_wait(1 - b)
                gathers_start(1 - b)

            accumulate(g, b)

            @pl.when(g % FLUSH == FLUSH - 1)
            def _():
                first = pl.multiple_of(base + (g + 1 - FLUSH) * C, OUTROWS)
                pltpu.sync_copy(outv, out_hbm.at[pl.ds(first, OUTROWS)])
        return 0

    lax.fori_loop(0, NCH // 2, outer, 0)


@jax.jit
def _sc_bow(emb, xflat):
    mesh = plsc.VectorSubcoreMesh(core_axis_name="c", subcore_axis_name="s",
                                  num_cores=NC, num_subcores=NS)
    f = pl.kernel(
        _sc_bow_kernel,
        out_type=jax.ShapeDtypeStruct((B, D), jnp.float32),
        mesh=mesh,
        scratch_types=[
            pltpu.VMEM((R,), jnp.int32),
            pltpu.VMEM((R,), jnp.int32),
            pltpu.VMEM((R, D), jnp.float32),
            pltpu.VMEM((R, D), jnp.float32),
            pltpu.VMEM((OUTROWS, D), jnp.float32),
            pltpu.SemaphoreType.DMA,
            pltpu.SemaphoreType.DMA,
            pltpu.SemaphoreType.DMA,
            pltpu.SemaphoreType.DMA,
        ],
        compiler_params=pltpu.CompilerParams(use_tc_tiling_on_sc=False),
    )
    return f(emb, xflat)


def _tc_head_kernel(sums_ref, t_ref, W1_ref, b1_ref, gamma_ref, beta_ref,
                    W2_ref, b2_ref, logits_ref, loss_ref):
    bow = sums_ref[...] * (1.0 / L)
    h = lax.dot_general(bow, W1_ref[...], (((1,), (1,)), ((), ())),
                        preferred_element_type=jnp.float32) + b1_ref[...]
    mu = jnp.mean(h, axis=0, keepdims=True)
    d = h - mu
    var = jnp.mean(d * d, axis=0, keepdims=True)
    hn = d * lax.rsqrt(var + 1e-5) * gamma_ref[...] + beta_ref[...]
    hr = jnp.maximum(hn, 0.0)
    logits = lax.dot_general(hr, W2_ref[...], (((1,), (0,)), ((), ())),
                             preferred_element_type=jnp.float32) + b2_ref[...]
    logits_ref[...] = logits
    t = t_ref[...]
    lv = (jnp.maximum(logits, 0.0) - logits * t
          + jnp.log1p(jnp.exp(-jnp.abs(logits))))
    loss_ref[...] = jnp.broadcast_to(jnp.mean(lv), (1, 1))


@jax.jit
def _tc_head(sums, t, W1, b1, gamma, beta, W2, b2):
    return pl.pallas_call(
        _tc_head_kernel,
        out_shape=(
            jax.ShapeDtypeStruct((B, 1), jnp.float32),
            jax.ShapeDtypeStruct((1, 1), jnp.float32),
        ),
    )(sums, t.reshape(B, 1), W1, b1.reshape(1, D), gamma.reshape(1, D),
      beta.reshape(1, D), W2, b2.reshape(1, 1))


def kernel(x, t, emb, W1, b1, gamma, beta, W2, b2):
    xflat = x.astype(jnp.int32).reshape(B * L)
    sums = _sc_bow(emb, xflat)
    logits2, loss2 = _tc_head(sums, t, W1, b1, gamma, beta, W2, b2)
    return (loss2[0, 0], logits2[:, 0])


# speedup vs baseline: 1.1654x; 1.1622x over previous
import functools

import jax
import jax.numpy as jnp
from jax import lax
from jax.experimental import pallas as pl
from jax.experimental.pallas import tpu as pltpu
from jax.experimental.pallas import tpu_sc as plsc

B, L, V, D = 16384, 200, 1000000, 64
NC, NS = 2, 16
NW = NC * NS
BPW = B // NW
C = 2
R = C * L
NCH = BPW // C
OUTROWS = 64
FLUSH = OUTROWS // C
STRIP = R
STRIPS = []
_off = 0
while _off < R:
    n = min(STRIP, R - _off)
    STRIPS.append((_off, n))
    _off += n

_KCOLS = D // 16


NBUF = 4


def _sc_bow_kernel(emb_hbm, xflat_hbm, out_hbm,
                   idx0, idx1, idx2, idx3,
                   rows0, rows1, rows2, rows3, outv,
                   is0, is1, is2, is3, rs0, rs1, rs2, rs3):
    cid = lax.axis_index("c")
    sid = lax.axis_index("s")
    wid = sid * NC + cid
    base = wid * BPW

    idxb = [idx0, idx1, idx2, idx3]
    rowsb = [rows0, rows1, rows2, rows3]
    isem = [is0, is1, is2, is3]
    rsem = [rs0, rs1, rs2, rs3]

    def idx_start(g, b):
        start = pl.multiple_of((base + g * C) * L, R)
        pltpu.async_copy(xflat_hbm.at[pl.ds(start, R)], idxb[b], isem[b])

    def idx_wait(b):
        pltpu.make_async_copy(xflat_hbm.at[pl.ds(0, R)], idxb[b],
                              isem[b]).wait()

    def gathers_start(b):
        for (off, n) in STRIPS:
            pltpu.async_copy(emb_hbm.at[idxb[b].at[pl.ds(off, n)]],
                             rowsb[b].at[pl.ds(off, n)], rsem[b])

    def gathers_wait(b):
        for (off, n) in STRIPS:
            pltpu.make_async_copy(emb_hbm.at[idxb[b].at[pl.ds(off, n)]],
                                  rowsb[b].at[pl.ds(off, n)],
                                  rsem[b]).wait()

    def accumulate(g, b):
        rv = rowsb[b]
        zeros = tuple(jnp.zeros((16,), jnp.float32)
                      for _ in range(C * _KCOLS))

        @plsc.parallel_loop(0, L, step=1, unroll=4, carry=zeros)
        def acc(l, a):
            new = []
            for r in range(C):
                for k in range(_KCOLS):
                    v = rv[r * L + l, pl.ds(k * 16, 16)]
                    new.append(a[r * _KCOLS + k] + v)
            return tuple(new)
        orow0 = (g % FLUSH) * C
        for r in range(C):
            for k in range(_KCOLS):
                outv[orow0 + r, pl.ds(k * 16, 16)] = acc[r * _KCOLS + k]

    for j in range(NBUF - 1):
        idx_start(j, j)
        idx_wait(j)
        gathers_start(j)
    idx_start(NBUF - 1, NBUF - 1)

    def outer(i, _):
        for b in range(NBUF):
            g = i * NBUF + b
            gathers_wait(b)

            bn = (b + NBUF - 1) % NBUF

            @pl.when(g + NBUF - 1 < NCH)
            def _():
                idx_wait(bn)
                gathers_start(bn)

            @pl.when(g + NBUF < NCH)
            def _():
                idx_start(g + NBUF, b)

            accumulate(g, b)

            @pl.when(g % FLUSH == FLUSH - 1)
            def _():
                first = pl.multiple_of(base + (g + 1 - FLUSH) * C, OUTROWS)
                pltpu.sync_copy(outv, out_hbm.at[pl.ds(first, OUTROWS)])
        return 0

    lax.fori_loop(0, NCH // NBUF, outer, 0)


@jax.jit
def _sc_bow(emb, xflat):
    mesh = plsc.VectorSubcoreMesh(core_axis_name="c", subcore_axis_name="s",
                                  num_cores=NC, num_subcores=NS)
    f = pl.kernel(
        _sc_bow_kernel,
        out_type=jax.ShapeDtypeStruct((B, D), jnp.float32),
        mesh=mesh,
        scratch_types=(
            [pltpu.VMEM((R,), jnp.int32) for _ in range(NBUF)]
            + [pltpu.VMEM((R, D), jnp.float32) for _ in range(NBUF)]
            + [pltpu.VMEM((OUTROWS, D), jnp.float32)]
            + [pltpu.SemaphoreType.DMA for _ in range(2 * NBUF)]
        ),
        compiler_params=pltpu.CompilerParams(use_tc_tiling_on_sc=False),
    )
    return f(emb, xflat)


def _tc_head_kernel(sums_ref, t_ref, W1_ref, b1_ref, gamma_ref, beta_ref,
                    W2_ref, b2_ref, logits_ref, loss_ref):
    bow = sums_ref[...] * (1.0 / L)
    h = lax.dot_general(bow, W1_ref[...], (((1,), (1,)), ((), ())),
                        preferred_element_type=jnp.float32) + b1_ref[...]
    mu = jnp.mean(h, axis=0, keepdims=True)
    d = h - mu
    var = jnp.mean(d * d, axis=0, keepdims=True)
    hn = d * lax.rsqrt(var + 1e-5) * gamma_ref[...] + beta_ref[...]
    hr = jnp.maximum(hn, 0.0)
    logits = lax.dot_general(hr, W2_ref[...], (((1,), (0,)), ((), ())),
                             preferred_element_type=jnp.float32) + b2_ref[...]
    logits_ref[...] = logits
    t = t_ref[...]
    lv = (jnp.maximum(logits, 0.0) - logits * t
          + jnp.log1p(jnp.exp(-jnp.abs(logits))))
    loss_ref[...] = jnp.broadcast_to(jnp.mean(lv), (1, 1))


@jax.jit
def _tc_head(sums, t, W1, b1, gamma, beta, W2, b2):
    return pl.pallas_call(
        _tc_head_kernel,
        out_shape=(
            jax.ShapeDtypeStruct((B, 1), jnp.float32),
            jax.ShapeDtypeStruct((1, 1), jnp.float32),
        ),
    )(sums, t.reshape(B, 1), W1, b1.reshape(1, D), gamma.reshape(1, D),
      beta.reshape(1, D), W2, b2.reshape(1, 1))


def kernel(x, t, emb, W1, b1, gamma, beta, W2, b2):
    xflat = x.astype(jnp.int32).reshape(B * L)
    sums = _sc_bow(emb, xflat)
    logits2, loss2 = _tc_head(sums, t, W1, b1, gamma, beta, W2, b2)
    return (loss2[0, 0], logits2[:, 0])
